# baseline (speedup 1.0000x reference)
import functools

import jax
import jax.numpy as jnp
from jax import lax
from jax.experimental import pallas as pl
from jax.experimental.pallas import tpu as pltpu
from jax.experimental.pallas import tpu_sc as plsc

NC = 2
NS = 16
NW = NC * NS
K = 128
NB = 4


def _mesh():
  return plsc.VectorSubcoreMesh(
      core_axis_name="c", subcore_axis_name="s",
      num_cores=NC, num_subcores=NS)


def _make_deg_kernel(e_pad, n_acc):
  epw = e_pad // NW
  chunks = epw // K
  rpt = n_acc // NS

  @functools.partial(
      pl.kernel,
      out_type=jax.ShapeDtypeStruct((NC, n_acc), jnp.float32),
      mesh=_mesh(),
      scratch_types=[
          pltpu.VMEM_SHARED((n_acc,), jnp.float32),
          pltpu.VMEM((K,), jnp.float32),
          pltpu.VMEM((K,), jnp.float32),
          [pltpu.VMEM((K,), jnp.int32) for _ in range(2 * NB)],
          [pltpu.SemaphoreType.DMA for _ in range(2 * NB)],
          [pltpu.SemaphoreType.DMA for _ in range(NB)],
      ],
  )
  def deg_kernel(dst_h, deg_out, deg_sh, zrow, ones, didx, isems, sems):
    c = lax.axis_index("c")
    s = lax.axis_index("s")
    for j in range(K // 16):
      zrow[pl.ds(j * 16, 16)] = jnp.zeros((16,), jnp.float32)
      ones[pl.ds(j * 16, 16)] = jnp.ones((16,), jnp.float32)
    ebase = (c * NS + s) * epw
    for m in range(NB):
      pltpu.async_copy(dst_h.at[pl.ds(ebase + m * K, K)], didx[m],
                       isems[m])
    def zloop(r, carry):
      pltpu.sync_copy(zrow, deg_sh.at[pl.ds(s * rpt + r * K, K)])
      return carry
    lax.fori_loop(0, rpt // K, zloop, 0)
    plsc.subcore_barrier()
    def eloop(j, carry):
      for mm in range(2 * NB):
        k = 2 * NB * j + mm
        mi = mm
        ms = mm % NB
        mi4 = (mm + NB) % (2 * NB)
        @pl.when(k >= NB)
        def _drain():
          pltpu.make_async_copy(ones, deg_sh.at[didx[(mm - NB) % (2 * NB)]],
                                sems[ms]).wait()
        @pl.when(k + NB < chunks)
        def _refill():
          pltpu.async_copy(dst_h.at[pl.ds(ebase + (k + NB) * K, K)],
                           didx[mi4], isems[mi4])
        pltpu.make_async_copy(dst_h.at[pl.ds(ebase + k * K, K)], didx[mi],
                              isems[mi]).wait()
        pltpu.async_copy(ones, deg_sh.at[didx[mi]], sems[ms], add=True)
      return carry
    lax.fori_loop(0, chunks // (2 * NB), eloop, 0)
    for mm in range(NB):
      m_last = (chunks - NB + mm) % (2 * NB)
      pltpu.make_async_copy(ones, deg_sh.at[didx[m_last]],
                            sems[m_last % NB]).wait()
    plsc.subcore_barrier()
    pltpu.sync_copy(deg_sh.at[pl.ds(s * rpt, rpt)],
                    deg_out.at[c, pl.ds(s * rpt, rpt)])

  return deg_kernel


def _make_agg_kernel(e_pad, n_acc, ch):
  epw = e_pad // NW
  chunks = epw // K
  rpt = n_acc // NS

  @functools.partial(
      pl.kernel,
      out_type=jax.ShapeDtypeStruct((NC, n_acc, ch), jnp.float32),
      mesh=_mesh(),
      scratch_types=[
          pltpu.VMEM_SHARED((n_acc, ch), jnp.float32),
          pltpu.VMEM((16, ch), jnp.float32),
          [pltpu.VMEM((K,), jnp.int32) for _ in range(2 * NB)],
          [pltpu.VMEM((K,), jnp.int32) for _ in range(2 * NB)],
          [pltpu.VMEM((K, ch), jnp.float32) for _ in range(2)],
          [pltpu.SemaphoreType.DMA for _ in range(2 * NB)],
          [pltpu.SemaphoreType.DMA for _ in range(2 * NB)],
          [pltpu.SemaphoreType.DMA for _ in range(2)],
          [pltpu.SemaphoreType.DMA for _ in range(2)],
      ],
  )
  def agg_kernel(src_h, dst_h, xs_h, acc_out, acc_sh, zbuf,
                 sidx, didx, rows, isems_s, isems_d, gsems, ssems):
    c = lax.axis_index("c")
    s = lax.axis_index("s")
    for i in range(16):
      for j in range(ch // 16):
        zbuf[i, pl.ds(j * 16, 16)] = jnp.zeros((16,), jnp.float32)
    ebase = (c * NS + s) * epw
    for m in range(NB):
      pltpu.async_copy(src_h.at[pl.ds(ebase + m * K, K)], sidx[m],
                       isems_s[m])
      pltpu.async_copy(dst_h.at[pl.ds(ebase + m * K, K)], didx[m],
                       isems_d[m])
    def zloop(r, carry):
      pltpu.sync_copy(zbuf, acc_sh.at[pl.ds(s * rpt + r * 16, 16)])
      return carry
    lax.fori_loop(0, rpt // 16, zloop, 0)
    plsc.subcore_barrier()
    pltpu.make_async_copy(src_h.at[pl.ds(ebase, K)], sidx[0],
                          isems_s[0]).wait()
    pltpu.async_copy(xs_h.at[sidx[0]], rows[0], gsems[0])
    def mloop(j, carry):
      for mm in range(2 * NB):
        k = 2 * NB * j + mm
        mr = mm % 2
        mr1 = (mm + 1) % 2
        mi = mm
        mi4 = (mm + NB) % (2 * NB)
        mi1 = (mm + 1) % (2 * NB)
        mim1 = (mm - 1) % (2 * NB)
        @pl.when(k >= 1)
        def _drain():
          pltpu.make_async_copy(rows[mr1], acc_sh.at[didx[mim1]],
                                ssems[mr1]).wait()
        @pl.when(k + NB < chunks)
        def _refill():
          pltpu.async_copy(src_h.at[pl.ds(ebase + (k + NB) * K, K)],
                           sidx[mi4], isems_s[mi4])
          pltpu.async_copy(dst_h.at[pl.ds(ebase + (k + NB) * K, K)],
                           didx[mi4], isems_d[mi4])
        @pl.when(k + 1 < chunks)
        def _prefetch():
          pltpu.make_async_copy(src_h.at[pl.ds(ebase + (k + 1) * K, K)],
                                sidx[mi1], isems_s[mi1]).wait()
          pltpu.async_copy(xs_h.at[sidx[mi1]], rows[mr1], gsems[mr1])
        pltpu.make_async_copy(xs_h.at[sidx[mi]], rows[mr],
                              gsems[mr]).wait()
        pltpu.make_async_copy(dst_h.at[pl.ds(ebase + k * K, K)], didx[mi],
                              isems_d[mi]).wait()
        pltpu.async_copy(rows[mr], acc_sh.at[didx[mi]], ssems[mr],
                         add=True)
      return carry
    lax.fori_loop(0, chunks // (2 * NB), mloop, 0)
    kk = chunks - 1
    pltpu.make_async_copy(rows[kk % 2], acc_sh.at[didx[kk % (2 * NB)]],
                          ssems[kk % 2]).wait()
    plsc.subcore_barrier()
    def dloop(r, carry):
      rb = s * rpt + r * K
      pltpu.sync_copy(acc_sh.at[pl.ds(rb, K)], acc_out.at[c, pl.ds(rb, K)])
      return carry
    lax.fori_loop(0, rpt // K, dloop, 0)

  return agg_kernel


def _tca_body(n, x_ref, deg_ref, xs_ref, dis_ref):
  deg = deg_ref[0] + deg_ref[1] + 1.0
  dis = lax.rsqrt(deg[:n])
  xs_ref[...] = x_ref[...] * dis[:, None]
  dis_ref[...] = dis[:, None]


def _tcb_body(n, acc_ref, xs_ref, dis_ref, w_ref, b_ref, out_ref):
  a = (acc_ref[0, :n] + acc_ref[1, :n] + xs_ref[:n]) * dis_ref[...]
  out = lax.dot_general(a, w_ref[...],
                        dimension_numbers=(((1,), (1,)), ((), ())),
                        preferred_element_type=jnp.float32)
  out_ref[...] = jnp.maximum(out + b_ref[...], 0.0)


@jax.jit
def kernel(x, edge_index, W, b):
  n, in_ch = x.shape
  out_ch = W.shape[0]
  e = edge_index.shape[1]

  unit = NW * NB * K
  e_pad = ((e + unit - 1) // unit) * unit
  n_acc = ((n + NS * K - 1) // (NS * K)) * (NS * K)
  npad = e_pad - e
  src = edge_index[0].astype(jnp.int32)
  dst = edge_index[1].astype(jnp.int32)
  if npad:
    fill = jnp.arange(npad, dtype=jnp.int32)
    src = jnp.concatenate([src, fill % n])
    dst = jnp.concatenate([dst, n + fill % (n_acc - n)])

  deg_pair = _make_deg_kernel(e_pad, n_acc)(dst)

  xs, dis = pl.pallas_call(
      functools.partial(_tca_body, n),
      out_shape=[
          jax.ShapeDtypeStruct((n, in_ch), jnp.float32),
          jax.ShapeDtypeStruct((n, 1), jnp.float32),
      ],
  )(x, deg_pair)

  acc_pair = _make_agg_kernel(e_pad, n_acc, in_ch)(src, dst, xs)

  out = pl.pallas_call(
      functools.partial(_tcb_body, n),
      out_shape=jax.ShapeDtypeStruct((n, out_ch), jnp.float32),
  )(acc_pair, xs, dis, W, b.reshape(1, out_ch))
  return out


# speedup vs baseline: 1.0073x; 1.0073x over previous
import functools

import jax
import jax.numpy as jnp
from jax import lax
from jax.experimental import pallas as pl
from jax.experimental.pallas import tpu as pltpu
from jax.experimental.pallas import tpu_sc as plsc

NC = 2
NS = 16
NW = NC * NS
K = 128
NB = 4


def _mesh():
  return plsc.VectorSubcoreMesh(
      core_axis_name="c", subcore_axis_name="s",
      num_cores=NC, num_subcores=NS)


def _make_deg_kernel(e_pad, n_acc):
  epw = e_pad // NW
  chunks = epw // K
  rpt = n_acc // NS

  @functools.partial(
      pl.kernel,
      out_type=jax.ShapeDtypeStruct((NC, n_acc), jnp.float32),
      mesh=_mesh(),
      scratch_types=[
          pltpu.VMEM_SHARED((n_acc,), jnp.float32),
          pltpu.VMEM((K,), jnp.float32),
          pltpu.VMEM((K,), jnp.float32),
          [pltpu.VMEM((K,), jnp.int32) for _ in range(2 * NB)],
          [pltpu.SemaphoreType.DMA for _ in range(2 * NB)],
          [pltpu.SemaphoreType.DMA for _ in range(NB)],
      ],
  )
  def deg_kernel(dst_h, deg_out, deg_sh, zrow, ones, didx, isems, sems):
    c = lax.axis_index("c")
    s = lax.axis_index("s")
    for j in range(K // 16):
      zrow[pl.ds(j * 16, 16)] = jnp.zeros((16,), jnp.float32)
      ones[pl.ds(j * 16, 16)] = jnp.ones((16,), jnp.float32)
    ebase = (c * NS + s) * epw
    for m in range(NB):
      pltpu.async_copy(dst_h.at[pl.ds(ebase + m * K, K)], didx[m],
                       isems[m])
    def zloop(r, carry):
      pltpu.sync_copy(zrow, deg_sh.at[pl.ds(s * rpt + r * K, K)])
      return carry
    lax.fori_loop(0, rpt // K, zloop, 0)
    plsc.subcore_barrier()
    def eloop(j, carry):
      for mm in range(2 * NB):
        k = 2 * NB * j + mm
        mi = mm
        ms = mm % NB
        mi4 = (mm + NB) % (2 * NB)
        @pl.when(k >= NB)
        def _drain():
          pltpu.make_async_copy(ones, deg_sh.at[didx[(mm - NB) % (2 * NB)]],
                                sems[ms]).wait()
        @pl.when(k + NB < chunks)
        def _refill():
          pltpu.async_copy(dst_h.at[pl.ds(ebase + (k + NB) * K, K)],
                           didx[mi4], isems[mi4])
        pltpu.make_async_copy(dst_h.at[pl.ds(ebase + k * K, K)], didx[mi],
                              isems[mi]).wait()
        pltpu.async_copy(ones, deg_sh.at[didx[mi]], sems[ms], add=True)
      return carry
    lax.fori_loop(0, chunks // (2 * NB), eloop, 0)
    for mm in range(NB):
      m_last = (chunks - NB + mm) % (2 * NB)
      pltpu.make_async_copy(ones, deg_sh.at[didx[m_last]],
                            sems[m_last % NB]).wait()
    plsc.subcore_barrier()
    pltpu.sync_copy(deg_sh.at[pl.ds(s * rpt, rpt)],
                    deg_out.at[c, pl.ds(s * rpt, rpt)])

  return deg_kernel


def _make_agg_kernel(e_pad, n_acc, ch):
  epw = e_pad // NW
  chunks = epw // K
  rpt = n_acc // NS

  @functools.partial(
      pl.kernel,
      out_type=jax.ShapeDtypeStruct((NC, n_acc, ch), jnp.float32),
      mesh=_mesh(),
      scratch_types=[
          pltpu.VMEM_SHARED((n_acc, ch), jnp.float32),
          pltpu.VMEM((16, ch), jnp.float32),
          [pltpu.VMEM((K,), jnp.int32) for _ in range(2 * NB)],
          [pltpu.VMEM((K,), jnp.int32) for _ in range(2 * NB)],
          [pltpu.VMEM((K, ch), jnp.float32) for _ in range(2)],
          [pltpu.SemaphoreType.DMA for _ in range(2 * NB)],
          [pltpu.SemaphoreType.DMA for _ in range(2 * NB)],
          [pltpu.SemaphoreType.DMA for _ in range(2)],
          [pltpu.SemaphoreType.DMA for _ in range(2)],
          pltpu.SemaphoreType.DMA,
      ],
  )
  def agg_kernel(src_h, dst_h, xs_h, acc_out, acc_sh, zbuf,
                 sidx, didx, rows, isems_s, isems_d, gsems, ssems, zsem):
    c = lax.axis_index("c")
    s = lax.axis_index("s")
    for i in range(16):
      for j in range(ch // 16):
        zbuf[i, pl.ds(j * 16, 16)] = jnp.zeros((16,), jnp.float32)
    ebase = (c * NS + s) * epw
    for m in range(NB):
      pltpu.async_copy(src_h.at[pl.ds(ebase + m * K, K)], sidx[m],
                       isems_s[m])
      pltpu.async_copy(dst_h.at[pl.ds(ebase + m * K, K)], didx[m],
                       isems_d[m])
    def zloop(r, carry):
      pltpu.async_copy(zbuf, acc_sh.at[pl.ds(s * rpt + r * 16, 16)], zsem)
      return carry
    lax.fori_loop(0, rpt // 16, zloop, 0)
    def zdrain(r, carry):
      pltpu.make_async_copy(zbuf, acc_sh.at[pl.ds(s * rpt + r * 16, 16)],
                            zsem).wait()
      return carry
    lax.fori_loop(0, rpt // 16, zdrain, 0)
    plsc.subcore_barrier()
    pltpu.make_async_copy(src_h.at[pl.ds(ebase, K)], sidx[0],
                          isems_s[0]).wait()
    pltpu.async_copy(xs_h.at[sidx[0]], rows[0], gsems[0])
    def mloop(j, carry):
      for mm in range(2 * NB):
        k = 2 * NB * j + mm
        mr = mm % 2
        mr1 = (mm + 1) % 2
        mi = mm
        mi4 = (mm + NB) % (2 * NB)
        mi1 = (mm + 1) % (2 * NB)
        mim1 = (mm - 1) % (2 * NB)
        @pl.when(k >= 1)
        def _drain():
          pltpu.make_async_copy(rows[mr1], acc_sh.at[didx[mim1]],
                                ssems[mr1]).wait()
        @pl.when(k + NB < chunks)
        def _refill():
          pltpu.async_copy(src_h.at[pl.ds(ebase + (k + NB) * K, K)],
                           sidx[mi4], isems_s[mi4])
          pltpu.async_copy(dst_h.at[pl.ds(ebase + (k + NB) * K, K)],
                           didx[mi4], isems_d[mi4])
        @pl.when(k + 1 < chunks)
        def _prefetch():
          pltpu.make_async_copy(src_h.at[pl.ds(ebase + (k + 1) * K, K)],
                                sidx[mi1], isems_s[mi1]).wait()
          pltpu.async_copy(xs_h.at[sidx[mi1]], rows[mr1], gsems[mr1])
        pltpu.make_async_copy(xs_h.at[sidx[mi]], rows[mr],
                              gsems[mr]).wait()
        pltpu.make_async_copy(dst_h.at[pl.ds(ebase + k * K, K)], didx[mi],
                              isems_d[mi]).wait()
        pltpu.async_copy(rows[mr], acc_sh.at[didx[mi]], ssems[mr],
                         add=True)
      return carry
    lax.fori_loop(0, chunks // (2 * NB), mloop, 0)
    kk = chunks - 1
    pltpu.make_async_copy(rows[kk % 2], acc_sh.at[didx[kk % (2 * NB)]],
                          ssems[kk % 2]).wait()
    plsc.subcore_barrier()
    def dloop(r, carry):
      rb = s * rpt + r * K
      pltpu.async_copy(acc_sh.at[pl.ds(rb, K)], acc_out.at[c, pl.ds(rb, K)],
                       zsem)
      return carry
    lax.fori_loop(0, rpt // K, dloop, 0)
    def ddrain(r, carry):
      rb = s * rpt + r * K
      pltpu.make_async_copy(acc_sh.at[pl.ds(rb, K)],
                            acc_out.at[c, pl.ds(rb, K)], zsem).wait()
      return carry
    lax.fori_loop(0, rpt // K, ddrain, 0)

  return agg_kernel


def _tca_body(n, x_ref, deg_ref, xs_ref, dis_ref):
  deg = deg_ref[0] + deg_ref[1] + 1.0
  dis = lax.rsqrt(deg[:n])
  xs_ref[...] = x_ref[...] * dis[:, None]
  dis_ref[...] = dis[:, None]


def _tcb_body(n, acc_ref, xs_ref, dis_ref, w_ref, b_ref, out_ref):
  a = (acc_ref[0, :n] + acc_ref[1, :n] + xs_ref[:n]) * dis_ref[...]
  out = lax.dot_general(a, w_ref[...],
                        dimension_numbers=(((1,), (1,)), ((), ())),
                        preferred_element_type=jnp.float32)
  out_ref[...] = jnp.maximum(out + b_ref[...], 0.0)


@jax.jit
def kernel(x, edge_index, W, b):
  n, in_ch = x.shape
  out_ch = W.shape[0]
  e = edge_index.shape[1]

  unit = NW * NB * K
  e_pad = ((e + unit - 1) // unit) * unit
  n_acc = ((n + NS * K - 1) // (NS * K)) * (NS * K)
  npad = e_pad - e
  src = edge_index[0].astype(jnp.int32)
  dst = edge_index[1].astype(jnp.int32)
  if npad:
    fill = jnp.arange(npad, dtype=jnp.int32)
    src = jnp.concatenate([src, fill % n])
    dst = jnp.concatenate([dst, n + fill % (n_acc - n)])

  deg_pair = _make_deg_kernel(e_pad, n_acc)(dst)

  xs, dis = pl.pallas_call(
      functools.partial(_tca_body, n),
      out_shape=[
          jax.ShapeDtypeStruct((n, in_ch), jnp.float32),
          jax.ShapeDtypeStruct((n, 1), jnp.float32),
      ],
  )(x, deg_pair)

  acc_pair = _make_agg_kernel(e_pad, n_acc, in_ch)(src, dst, xs)

  out = pl.pallas_call(
      functools.partial(_tcb_body, n),
      out_shape=jax.ShapeDtypeStruct((n, out_ch), jnp.float32),
  )(acc_pair, xs, dis, W, b.reshape(1, out_ch))
  return out
